# baseline (speedup 1.0000x reference)
import functools

import jax
import jax.numpy as jnp
from jax import lax
from jax.experimental import pallas as pl
from jax.experimental.pallas import tpu as pltpu
from jax.experimental.pallas import tpu_sc as plsc

N = 10000
D = 128
DP = 144
NACC = 10016
E = 320000
NC, NS = 2, 16
NW = NC * NS
KC = 104
NT = 100
EPT = NT * KC
NHALF = 2
NTH = NT // NHALF
NBUF = 2
NGRP = NTH // NBUF
ZROWS = NACC // NS


def _sc_body(xa_hbm, src_hbm, dst_hbm, out_hbm,
             src_v, dst_v, rows, acc, *sems):
    c = lax.axis_index("c")
    s = lax.axis_index("s")
    wid = s * NC + c
    gsems = sems[:NBUF]
    ssems = sems[NBUF:]

    zb = rows.at[0]
    def zrow(i, carry):
        r = i // (DP // 16)
        col = (i % (DP // 16)) * 16
        zb[r, pl.ds(col, 16)] = jnp.zeros((16,), jnp.float32)
        return carry
    lax.fori_loop(0, KC * DP // 16, zrow, 0)
    row0 = s * ZROWS
    nfull = ZROWS // KC
    for j in range(nfull):
        pltpu.sync_copy(zb, acc.at[pl.ds(row0 + j * KC, KC)])
    rem = ZROWS - nfull * KC
    if rem:
        pltpu.sync_copy(zb.at[pl.ds(0, rem)], acc.at[pl.ds(row0 + nfull * KC, rem)])
    plsc.subcore_barrier()

    def fire_gather(g, b):
        pltpu.async_copy(
            xa_hbm.at[src_v.at[pl.ds(g * KC, KC)]], rows.at[b], gsems[b])

    def wait_gather(g, b):
        pltpu.make_async_copy(
            xa_hbm.at[src_v.at[pl.ds(g * KC, KC)]], rows.at[b], gsems[b]).wait()

    def fire_scatter(g, b):
        pltpu.async_copy(rows.at[b], acc.at[dst_v.at[g]], ssems[b], add=True)

    def wait_scatter(g, b):
        pltpu.make_async_copy(rows.at[b], acc.at[dst_v.at[g]], ssems[b]).wait()

    for h in range(NHALF):
        pltpu.sync_copy(src_hbm.at[wid, pl.ds(h * NTH * KC, NTH * KC)], src_v)
        pltpu.sync_copy(dst_hbm.at[wid, pl.ds(h * NTH, NTH)], dst_v)

        for b in range(NBUF):
            fire_gather(b, b)

        def group(gi, carry):
            for b in range(NBUF):
                g = gi * NBUF + b
                wait_gather(g, b)
                fire_scatter(g, b)
                wait_scatter(g, b)
                fire_gather(g + NBUF, b)
            return carry
        lax.fori_loop(0, NGRP - 1, group, 0)

        for b in range(NBUF):
            g = (NGRP - 1) * NBUF + b
            wait_gather(g, b)
            fire_scatter(g, b)
            wait_scatter(g, b)

    plsc.subcore_barrier()

    pltpu.sync_copy(acc.at[pl.ds(row0, ZROWS)], out_hbm.at[c, pl.ds(row0, ZROWS)])


@functools.cache
def _sc_scatter():
    return pl.kernel(
        _sc_body,
        out_type=jax.ShapeDtypeStruct((NC, NACC, DP), jnp.float32),
        mesh=plsc.VectorSubcoreMesh(
            core_axis_name="c", subcore_axis_name="s",
            num_cores=NC, num_subcores=NS),
        scratch_types=[
            pltpu.VMEM((NTH * KC,), jnp.int32),
            pltpu.VMEM((NTH, KC), jnp.int32),
            pltpu.VMEM((NBUF, KC, DP), jnp.float32),
            pltpu.VMEM_SHARED((NACC, DP), jnp.float32),
        ] + [pltpu.SemaphoreType.DMA] * (2 * NBUF),
        compiler_params=pltpu.CompilerParams(use_tc_tiling_on_sc=False),
    )


BN = 1000


def _tc_body(x_ref, p_ref, w1a_ref, w1b_ref, w2a_ref, w2b_ref, o_ref):
    xb = x_ref[...]
    p0 = p_ref[0]
    p1 = p_ref[1]
    dg = p0[:, D:D + 1] + p1[:, D:D + 1]
    agg = (jnp.dot(p0[:, :D] + p1[:, :D], w1a_ref[...],
                   preferred_element_type=jnp.float32)
           + jnp.dot(xb * dg, w1b_ref[...], preferred_element_type=jnp.float32))
    agg = agg * jnp.float32(1.0 / 9.0)
    o_ref[...] = (
        jnp.dot(jnp.maximum(xb, 0.0), w2a_ref[...], preferred_element_type=jnp.float32)
        + jnp.dot(jnp.maximum(agg, 0.0), w2b_ref[...], preferred_element_type=jnp.float32))


def _tc_finish(x, p, w1a, w1b, w2a, w2b):
    wspec = pl.BlockSpec((D, D), lambda i: (0, 0))
    return pl.pallas_call(
        _tc_body,
        grid=(N // BN,),
        in_specs=[
            pl.BlockSpec((BN, D), lambda i: (i, 0)),
            pl.BlockSpec((NC, BN, DP), lambda i: (0, i, 0)),
            wspec, wspec, wspec, wspec,
        ],
        out_specs=pl.BlockSpec((BN, D), lambda i: (i, 0)),
        out_shape=jax.ShapeDtypeStruct((N, D), jnp.float32),
    )(x, p, w1a, w1b, w2a, w2b)


def kernel(x, edge_index, W1, W2):
    src = edge_index[:, 0].astype(jnp.int32)
    dst = edge_index[:, 1].astype(jnp.int32)
    pad = EPT - E // NW
    src_p = jnp.concatenate(
        [src.reshape(NW, E // NW), jnp.zeros((NW, pad), jnp.int32)], axis=1)
    dst_p = jnp.concatenate(
        [dst.reshape(NW, E // NW), jnp.full((NW, pad), N, jnp.int32)],
        axis=1).reshape(NW, NT, KC)
    xa = jnp.concatenate(
        [x, jnp.ones((N, 1), jnp.float32), jnp.zeros((N, DP - D - 1), jnp.float32)],
        axis=1)
    p = _sc_scatter()(xa, src_p, dst_p)
    return _tc_finish(x, p, W1[:D], W1[D:], W2[:D], W2[D:])


# speedup vs baseline: 1.8225x; 1.8225x over previous
import functools

import jax
import jax.numpy as jnp
from jax import lax
from jax.experimental import pallas as pl
from jax.experimental.pallas import tpu as pltpu
from jax.experimental.pallas import tpu_sc as plsc

N = 10000
D = 128
WL = 80
NACC = 10112
E = 320000
NC, NS = 2, 16
EPT = 20480
CH = 128
NCH = EPT // CH
NHALF = 2
HCH = NCH // NHALF
NBUF = 5
NGRP = HCH // NBUF
ZROWS = NACC // NS


def _sc_body(xlo_hbm, xhi_hbm, src_hbm, dst_hbm, out_hbm,
             src_v, dst_v, rows, acc, *sems):
    c = lax.axis_index("c")
    s = lax.axis_index("s")
    gsems = sems[:NBUF]
    ssems = sems[NBUF:]

    zb = rows.at[0]
    def zrow(i, carry):
        r = i // (WL // 16)
        col = (i % (WL // 16)) * 16
        zb[r, pl.ds(col, 16)] = jnp.zeros((16,), jnp.float32)
        return carry
    lax.fori_loop(0, CH * WL // 16, zrow, 0)
    row0 = s * ZROWS
    nfull = ZROWS // CH
    for j in range(nfull):
        pltpu.sync_copy(zb, acc.at[pl.ds(row0 + j * CH, CH)])
    rem = ZROWS - nfull * CH
    if rem:
        pltpu.sync_copy(zb.at[pl.ds(0, rem)], acc.at[pl.ds(row0 + nfull * CH, rem)])
    plsc.subcore_barrier()

    def run_edges(table):
        def fire_gather(g, b):
            pltpu.async_copy(
                table.at[src_v.at[pl.ds(g * CH, CH)]], rows.at[b], gsems[b])

        def wait_gather(g, b):
            pltpu.make_async_copy(
                table.at[src_v.at[pl.ds(g * CH, CH)]], rows.at[b], gsems[b]).wait()

        def fire_scatter(g, b):
            pltpu.async_copy(rows.at[b], acc.at[dst_v.at[g]], ssems[b], add=True)

        def wait_scatter(g, b):
            pltpu.make_async_copy(rows.at[b], acc.at[dst_v.at[g]], ssems[b]).wait()

        for h in range(NHALF):
            pltpu.sync_copy(src_hbm.at[s, pl.ds(h * HCH * CH, HCH * CH)], src_v)
            pltpu.sync_copy(dst_hbm.at[s, pl.ds(h * HCH, HCH)], dst_v)

            for b in range(NBUF):
                fire_gather(b, b)

            def group(gi, carry):
                for b in range(NBUF):
                    g = gi * NBUF + b
                    wait_gather(g, b)
                    fire_scatter(g, b)
                    wait_scatter(g, b)
                    fire_gather(g + NBUF, b)
                return carry
            lax.fori_loop(0, NGRP - 1, group, 0)

            for b in range(NBUF):
                g = (NGRP - 1) * NBUF + b
                wait_gather(g, b)
                fire_scatter(g, b)
                wait_scatter(g, b)

    @pl.when(c == 0)
    def _():
        run_edges(xlo_hbm)

    @pl.when(c == 1)
    def _():
        run_edges(xhi_hbm)

    plsc.subcore_barrier()

    pltpu.sync_copy(acc.at[pl.ds(row0, ZROWS)], out_hbm.at[c, pl.ds(row0, ZROWS)])


@functools.cache
def _sc_scatter():
    return pl.kernel(
        _sc_body,
        out_type=jax.ShapeDtypeStruct((NC, NACC, WL), jnp.float32),
        mesh=plsc.VectorSubcoreMesh(
            core_axis_name="c", subcore_axis_name="s",
            num_cores=NC, num_subcores=NS),
        scratch_types=[
            pltpu.VMEM((HCH * CH,), jnp.int32),
            pltpu.VMEM((HCH, CH), jnp.int32),
            pltpu.VMEM((NBUF, CH, WL), jnp.float32),
            pltpu.VMEM_SHARED((NACC, WL), jnp.float32),
        ] + [pltpu.SemaphoreType.DMA] * (2 * NBUF),
        compiler_params=pltpu.CompilerParams(use_tc_tiling_on_sc=False),
    )


BN = 1000


def _tc_body(x_ref, plo_ref, phi_ref, w1al_ref, w1ah_ref, w1b_ref,
             w2a_ref, w2b_ref, o_ref):
    xb = x_ref[...]
    plo = plo_ref[...]
    phi = phi_ref[...]
    dg = phi[:, D - WL:D - WL + 1]
    agg = (jnp.dot(plo, w1al_ref[...], preferred_element_type=jnp.float32)
           + jnp.dot(phi[:, :D - WL], w1ah_ref[...], preferred_element_type=jnp.float32)
           + jnp.dot(xb * dg, w1b_ref[...], preferred_element_type=jnp.float32))
    agg = agg * jnp.float32(1.0 / 9.0)
    o_ref[...] = (
        jnp.dot(jnp.maximum(xb, 0.0), w2a_ref[...], preferred_element_type=jnp.float32)
        + jnp.dot(jnp.maximum(agg, 0.0), w2b_ref[...], preferred_element_type=jnp.float32))


def _tc_finish(x, plo, phi, w1al, w1ah, w1b, w2a, w2b):
    def wspec(k):
        return pl.BlockSpec((k, D), lambda i: (0, 0))
    return pl.pallas_call(
        _tc_body,
        grid=(N // BN,),
        in_specs=[
            pl.BlockSpec((BN, D), lambda i: (i, 0)),
            pl.BlockSpec((BN, WL), lambda i: (i, 0)),
            pl.BlockSpec((BN, WL), lambda i: (i, 0)),
            wspec(WL), wspec(D - WL), wspec(D), wspec(D), wspec(D),
        ],
        out_specs=pl.BlockSpec((BN, D), lambda i: (i, 0)),
        out_shape=jax.ShapeDtypeStruct((N, D), jnp.float32),
    )(x, plo, phi, w1al, w1ah, w1b, w2a, w2b)


def kernel(x, edge_index, W1, W2):
    src = edge_index[:, 0].astype(jnp.int32)
    dst = edge_index[:, 1].astype(jnp.int32)
    pad = EPT - E // NS
    src_p = jnp.concatenate(
        [src.reshape(NS, E // NS), jnp.zeros((NS, pad), jnp.int32)], axis=1)
    dst_p = jnp.concatenate(
        [dst.reshape(NS, E // NS), jnp.full((NS, pad), N, jnp.int32)],
        axis=1).reshape(NS, NCH, CH)
    xa = jnp.concatenate(
        [x, jnp.ones((N, 1), jnp.float32),
         jnp.zeros((N, 2 * WL - D - 1), jnp.float32)], axis=1)
    partials = _sc_scatter()(xa[:, :WL], xa[:, WL:], src_p, dst_p)
    return _tc_finish(x, partials[0], partials[1],
                      W1[:WL], W1[WL:D], W1[D:], W2[:D], W2[D:])


# speedup vs baseline: 4.0804x; 2.2390x over previous
import functools

import jax
import jax.numpy as jnp
from jax import lax
from jax.experimental import pallas as pl
from jax.experimental.pallas import tpu as pltpu
from jax.experimental.pallas import tpu_sc as plsc

N = 10000
D = 128
WL = 64
WD = 16
NACC = 10016
E = 320000
NC, NS = 2, 16
KC = 80
NT = 250
EPT = NT * KC
NBUF = 4
NGRPF = 61
ZROWS = NACC // NS


def _sc_body(x2_hbm, src_hbm, dst_hbm, out_hbm, outdeg_hbm,
             src_v, dst_v, rows, ones_v, acc, accdeg, *sems):
    c = lax.axis_index("c")
    s = lax.axis_index("s")
    gsems = sems[:NBUF]
    ssems = sems[NBUF:2 * NBUF]
    dsem = sems[2 * NBUF]

    e0 = jnp.where(lax.iota(jnp.int32, 16) == 0, 1.0, 0.0).astype(jnp.float32)
    def orow(i, carry):
        ones_v[i, pl.ds(0, 16)] = e0
        return carry
    lax.fori_loop(0, KC, orow, 0)

    zb = rows.at[0]
    def zrow(i, carry):
        r = i // (WL // 16)
        col = (i % (WL // 16)) * 16
        zb[r, pl.ds(col, 16)] = jnp.zeros((16,), jnp.float32)
        return carry
    lax.fori_loop(0, KC * WL // 16, zrow, 0)
    row0 = s * ZROWS
    nfull = ZROWS // KC
    for j in range(nfull):
        pltpu.sync_copy(zb, acc.at[pl.ds(row0 + j * KC, KC)])
        pltpu.sync_copy(zb.at[pl.ds(0, KC), pl.ds(0, WD)],
                        accdeg.at[pl.ds(row0 + j * KC, KC)])
    rem = ZROWS - nfull * KC
    if rem:
        pltpu.sync_copy(zb.at[pl.ds(0, rem)], acc.at[pl.ds(row0 + nfull * KC, rem)])
        pltpu.sync_copy(zb.at[pl.ds(0, rem), pl.ds(0, WD)],
                        accdeg.at[pl.ds(row0 + nfull * KC, rem)])
    plsc.subcore_barrier()

    pltpu.sync_copy(src_hbm.at[s], src_v)
    pltpu.sync_copy(dst_hbm.at[s], dst_v)
    def remap(i, carry):
        v = src_v[pl.ds(i * 16, 16)]
        src_v[pl.ds(i * 16, 16)] = v + v + c
        return carry
    lax.fori_loop(0, EPT // 16, remap, 0)

    def fire_gather(g, b):
        pltpu.async_copy(
            x2_hbm.at[src_v.at[pl.ds(g * KC, KC)]], rows.at[b], gsems[b])

    def wait_gather(g, b):
        pltpu.make_async_copy(
            x2_hbm.at[src_v.at[pl.ds(g * KC, KC)]], rows.at[b], gsems[b]).wait()

    def fire_scatter(g, b, par):
        pltpu.async_copy(rows.at[b], acc.at[dst_v.at[g]], ssems[b], add=True)
        @pl.when(c == par)
        def _():
            pltpu.async_copy(ones_v, accdeg.at[dst_v.at[g]], dsem, add=True)

    def wait_scatter(g, b, par):
        pltpu.make_async_copy(rows.at[b], acc.at[dst_v.at[g]], ssems[b]).wait()
        @pl.when(c == par)
        def _():
            pltpu.make_async_copy(ones_v, accdeg.at[dst_v.at[g]], dsem).wait()

    for b in range(NBUF):
        fire_gather(b, b)

    def group(gi, carry):
        for b in range(NBUF):
            g = gi * NBUF + b
            wait_gather(g, b)
            fire_scatter(g, b, b % 2)
            wait_scatter(g, b, b % 2)
            fire_gather(g + NBUF, b)
        return carry
    lax.fori_loop(0, NGRPF, group, 0)

    for g in range(NGRPF * NBUF, NT):
        b = g % NBUF
        wait_gather(g, b)
        fire_scatter(g, b, g % 2)
        wait_scatter(g, b, g % 2)
        if g + NBUF < NT:
            fire_gather(g + NBUF, b)

    plsc.subcore_barrier()

    pltpu.sync_copy(acc.at[pl.ds(row0, ZROWS)], out_hbm.at[c, pl.ds(row0, ZROWS)])
    pltpu.sync_copy(accdeg.at[pl.ds(row0, ZROWS)],
                    outdeg_hbm.at[c, pl.ds(row0, ZROWS)])


@functools.cache
def _sc_scatter():
    return pl.kernel(
        _sc_body,
        out_type=(jax.ShapeDtypeStruct((NC, NACC, WL), jnp.float32),
                  jax.ShapeDtypeStruct((NC, NACC, WD), jnp.float32)),
        mesh=plsc.VectorSubcoreMesh(
            core_axis_name="c", subcore_axis_name="s",
            num_cores=NC, num_subcores=NS),
        scratch_types=[
            pltpu.VMEM((EPT,), jnp.int32),
            pltpu.VMEM((NT, KC), jnp.int32),
            pltpu.VMEM((NBUF, KC, WL), jnp.float32),
            pltpu.VMEM((KC, WD), jnp.float32),
            pltpu.VMEM_SHARED((NACC, WL), jnp.float32),
            pltpu.VMEM_SHARED((NACC, WD), jnp.float32),
        ] + [pltpu.SemaphoreType.DMA] * (2 * NBUF + 1),
        compiler_params=pltpu.CompilerParams(use_tc_tiling_on_sc=False),
    )


BN = 1000


def _tc_body(x_ref, p_ref, pd_ref, w1al_ref, w1ah_ref, w1b_ref,
             w2a_ref, w2b_ref, o_ref):
    xb = x_ref[...]
    dg = pd_ref[0, :, 0:1] + pd_ref[1, :, 0:1]
    agg = (jnp.dot(p_ref[0], w1al_ref[...], preferred_element_type=jnp.float32)
           + jnp.dot(p_ref[1], w1ah_ref[...], preferred_element_type=jnp.float32)
           + jnp.dot(xb * dg, w1b_ref[...], preferred_element_type=jnp.float32))
    agg = agg * jnp.float32(1.0 / 9.0)
    o_ref[...] = (
        jnp.dot(jnp.maximum(xb, 0.0), w2a_ref[...], preferred_element_type=jnp.float32)
        + jnp.dot(jnp.maximum(agg, 0.0), w2b_ref[...], preferred_element_type=jnp.float32))


def _tc_finish(x, p, pd, w1al, w1ah, w1b, w2a, w2b):
    def wspec(k):
        return pl.BlockSpec((k, D), lambda i: (0, 0))
    return pl.pallas_call(
        _tc_body,
        grid=(N // BN,),
        in_specs=[
            pl.BlockSpec((BN, D), lambda i: (i, 0)),
            pl.BlockSpec((NC, BN, WL), lambda i: (0, i, 0)),
            pl.BlockSpec((NC, BN, WD), lambda i: (0, i, 0)),
            wspec(WL), wspec(WL), wspec(D), wspec(D), wspec(D),
        ],
        out_specs=pl.BlockSpec((BN, D), lambda i: (i, 0)),
        out_shape=jax.ShapeDtypeStruct((N, D), jnp.float32),
    )(x, p, pd, w1al, w1ah, w1b, w2a, w2b)


def kernel(x, edge_index, W1, W2):
    ei = edge_index.astype(jnp.int32).T
    src_p = ei[0].reshape(NS, EPT)
    dst_p = ei[1].reshape(NS, NT, KC)
    x2 = x.reshape(2 * N, WL)
    p, pd = _sc_scatter()(x2, src_p, dst_p)
    return _tc_finish(x, p, pd,
                      W1[:WL], W1[WL:D], W1[D:], W2[:D], W2[D:])
